# Initial kernel scaffold; baseline (speedup 1.0000x reference)
import jax
import jax.numpy as jnp
from jax.experimental import pallas as pl


def kernel(input, target):
    raise NotImplementedError("write your pallas kernel here")


# speedup vs baseline: 11.2577x; 11.2577x over previous
import jax
import jax.numpy as jnp
from jax.experimental import pallas as pl

_NBISECT = 22
_NREFINE = 1


def _loss_block(x_ref, t_ref, out_ref):
    x = x_ref[...]
    tgt = t_ref[...]
    m = jnp.max(x, axis=1, keepdims=True)
    xs = (x - m) * 0.5

    lo = jnp.full_like(m, -1.0)
    hi = jnp.zeros_like(m)
    for _ in range(_NBISECT):
        mid = (lo + hi) * 0.5
        r = jnp.maximum(xs - mid, 0.0)
        f = jnp.sum(r * r, axis=1, keepdims=True)
        gt = f > 1.0
        lo = jnp.where(gt, mid, lo)
        hi = jnp.where(gt, hi, mid)
    tau = (lo + hi) * 0.5

    for _ in range(_NREFINE):
        mask = xs > tau
        one = jnp.ones_like(xs)
        k = jnp.sum(jnp.where(mask, one, 0.0), axis=1, keepdims=True)
        s1 = jnp.sum(jnp.where(mask, xs, 0.0), axis=1, keepdims=True)
        s2 = jnp.sum(jnp.where(mask, xs * xs, 0.0), axis=1, keepdims=True)
        mean = s1 / k
        delta = (1.0 - (s2 - s1 * mean)) / k
        tau = mean - jnp.sqrt(jnp.maximum(delta, 0.0))

    r = jnp.maximum(xs - tau, 0.0)
    p = r * r
    s3 = jnp.sum(p * r, axis=1, keepdims=True)
    iota = jax.lax.broadcasted_iota(jnp.int32, x.shape, 1)
    onehot = (iota == tgt).astype(x.dtype)
    spx = jnp.sum((p - onehot) * x, axis=1, keepdims=True)
    loss = (1.0 - s3) * (1.0 / 0.75) + spx
    out_ref[...] = jnp.reshape(jnp.sum(loss), (1, 1, 1))


def kernel(input, target):
    n, c = input.shape
    rows = 256 if n % 256 == 0 else n
    grid = n // rows
    tgt = target.astype(jnp.int32).reshape(n, 1)
    partials = pl.pallas_call(
        _loss_block,
        grid=(grid,),
        in_specs=[
            pl.BlockSpec((rows, c), lambda i: (i, 0)),
            pl.BlockSpec((rows, 1), lambda i: (i, 0)),
        ],
        out_specs=pl.BlockSpec((1, 1, 1), lambda i: (i, 0, 0)),
        out_shape=jax.ShapeDtypeStruct((grid, 1, 1), jnp.float32),
    )(input, tgt)
    return jnp.sum(partials) / float(n)


# speedup vs baseline: 16.7441x; 1.4874x over previous
import jax
import jax.numpy as jnp
from jax.experimental import pallas as pl
from jax.experimental.pallas import tpu as pltpu

_NBISECT = 8
_NREFINE = 2


def _loss_block(x_ref, t_ref, out_ref):
    x = x_ref[...]
    tgt = t_ref[...]
    m = jnp.max(x, axis=1, keepdims=True)
    xs = (x - m) * 0.5

    lo = jnp.full_like(m, -1.0)
    hi = jnp.zeros_like(m)
    for _ in range(_NBISECT):
        mid = (lo + hi) * 0.5
        r = jnp.maximum(xs - mid, 0.0)
        f = jnp.sum(r * r, axis=1, keepdims=True)
        gt = f > 1.0
        lo = jnp.where(gt, mid, lo)
        hi = jnp.where(gt, hi, mid)
    tau = (lo + hi) * 0.5

    for _ in range(_NREFINE):
        mask = xs > tau
        one = jnp.ones_like(xs)
        k = jnp.sum(jnp.where(mask, one, 0.0), axis=1, keepdims=True)
        s1 = jnp.sum(jnp.where(mask, xs, 0.0), axis=1, keepdims=True)
        s2 = jnp.sum(jnp.where(mask, xs * xs, 0.0), axis=1, keepdims=True)
        mean = s1 / k
        delta = (1.0 - (s2 - s1 * mean)) / k
        tau = mean - jnp.sqrt(jnp.maximum(delta, 0.0))

    r = jnp.maximum(xs - tau, 0.0)
    p = r * r
    s3 = jnp.sum(p * r, axis=1, keepdims=True)
    iota = jax.lax.broadcasted_iota(jnp.int32, x.shape, 1)
    onehot = (iota == tgt).astype(x.dtype)
    spx = jnp.sum((p - onehot) * x, axis=1, keepdims=True)
    loss = (1.0 - s3) * (1.0 / 0.75) + spx
    out_ref[...] = jnp.reshape(jnp.sum(loss), (1, 1, 1))


def kernel(input, target):
    n, c = input.shape
    rows = 256 if n % 256 == 0 else n
    grid = n // rows
    tgt = target.astype(jnp.int32).reshape(n, 1)
    partials = pl.pallas_call(
        _loss_block,
        grid=(grid,),
        in_specs=[
            pl.BlockSpec((rows, c), lambda i: (i, 0)),
            pl.BlockSpec((rows, 1), lambda i: (i, 0)),
        ],
        out_specs=pl.BlockSpec((1, 1, 1), lambda i: (i, 0, 0)),
        out_shape=jax.ShapeDtypeStruct((grid, 1, 1), jnp.float32),
        compiler_params=pltpu.CompilerParams(
            dimension_semantics=("parallel",),
        ),
    )(input, tgt)
    return jnp.sum(partials) / float(n)
